# baseline (speedup 1.0000x reference)
import jax
import jax.numpy as jnp
from jax.experimental import pallas as pl
from jax.experimental.pallas import tpu as pltpu

B, N, K = 8, 1024, 40
EPS = 1e-5
T = 128
NT = N // T
NEG = -jnp.inf

F32 = jnp.float32
HI = jax.lax.Precision.HIGHEST


def _lrelu(x):
    return jnp.where(x >= 0, x, 0.2 * x)


def _bf16(x):
    return x.astype(jnp.bfloat16).astype(F32)


def _knn_kernel(x_ref, idx_ref, d_ref):
    x = x_ref[0]
    xb = _bf16(x)
    xbT = xb.T
    G = xbT[:, 0:1] * xb[0:1, :]
    G = G + xbT[:, 1:2] * xb[1:2, :]
    G = G + xbT[:, 2:3] * xb[2:3, :]
    inner = -2.0 * G
    x2 = x * x
    xx_lane = jnp.sum(x2, axis=0, keepdims=True)
    xT = x.T
    xx_sub = jnp.sum(xT * xT, axis=1, keepdims=True)
    d_ref[...] = ((0.0 - xx_sub) - inner) - xx_lane
    iota = jax.lax.broadcasted_iota(jnp.int32, (N, N), 0)

    def body(k, _):
        D = d_ref[...]
        m = jnp.max(D, axis=0, keepdims=True)
        sel = jnp.min(jnp.where(D == m, iota, N), axis=0, keepdims=True)
        idx_ref[0, pl.ds(k, 1), :] = sel
        d_ref[...] = jnp.where(iota == sel, NEG, D)
        return 0

    jax.lax.fori_loop(0, K, body, 0, unroll=False)


def _knn(x_s):
    return pl.pallas_call(
        _knn_kernel,
        out_shape=jax.ShapeDtypeStruct((B, K, N), jnp.int32),
        grid=(B,),
        in_specs=[pl.BlockSpec((1, 3, N), lambda b: (b, 0, 0))],
        out_specs=pl.BlockSpec((1, K, N), lambda b: (b, 0, 0)),
        scratch_shapes=[pltpu.VMEM((N, N), F32)],
    )(x_s)


def _bn_coef(st, g, bt, cnt):
    mean = st[:, 0] / cnt
    var = st[:, 1] / cnt - mean * mean
    a = g / jnp.sqrt(var + EPS)
    b = bt - mean * a
    return a[:, None], b[:, None]


def _edge_conv1_kernel(x_ref, xt_ref, idx_ref, w1_ref, y1_ref, st_ref):
    b = pl.program_id(0)
    t = pl.program_id(1)
    x = x_ref[0]
    xt = xt_ref[0]
    W1 = w1_ref[...]
    cin = x.shape[0]

    def body(k, carry):
        s, q = carry
        ik = idx_ref[0, pl.ds(k, 1), :]
        xg = jnp.zeros((cin, T), F32)
        for g in range(N // 128):
            rel = ik - (g * 128)
            inb = (rel >= 0) & (rel < 128)
            idxb = jnp.broadcast_to(jnp.where(inb, rel, 0), (cin, T))
            gat = jnp.take_along_axis(x[:, g * 128:(g + 1) * 128],
                                      idxb, axis=1)
            xg = jnp.where(inb, gat, xg)
        ef = _bf16(jnp.concatenate([xg - xt, xt], axis=0))
        y1 = jnp.dot(W1, ef, preferred_element_type=F32)
        y1_ref[0, pl.ds(k, 1), :, :] = y1[None]
        return (s + jnp.sum(y1, axis=1, keepdims=True),
                q + jnp.sum(y1 * y1, axis=1, keepdims=True))

    s, q = jax.lax.fori_loop(0, K, body,
                             (jnp.zeros((64, 1), F32), jnp.zeros((64, 1), F32)),
                             unroll=False)
    st = jnp.concatenate([s, q], axis=1)

    @pl.when((b == 0) & (t == 0))
    def _():
        st_ref[...] = st

    @pl.when((b != 0) | (t != 0))
    def _():
        st_ref[...] += st


def _edge_conv1(x, idx, w1):
    cin = x.shape[1]
    return pl.pallas_call(
        _edge_conv1_kernel,
        out_shape=(
            jax.ShapeDtypeStruct((B, K, 64, N), F32),
            jax.ShapeDtypeStruct((64, 2), F32),
        ),
        grid=(B, NT),
        in_specs=[
            pl.BlockSpec((1, cin, N), lambda b, t: (b, 0, 0)),
            pl.BlockSpec((1, cin, T), lambda b, t: (b, 0, t)),
            pl.BlockSpec((1, K, T), lambda b, t: (b, 0, t)),
            pl.BlockSpec(w1.shape, lambda b, t: (0, 0)),
        ],
        out_specs=(
            pl.BlockSpec((1, K, 64, T), lambda b, t: (b, 0, 0, t)),
            pl.BlockSpec((64, 2), lambda b, t: (0, 0)),
        ),
    )(x, x, idx, w1)


def _bn_conv2_kernel(y1_ref, a1_ref, b1_ref, w2_ref, y2_ref, st_ref):
    b = pl.program_id(0)
    t = pl.program_id(1)
    a1 = a1_ref[...]
    b1 = b1_ref[...]
    W2 = w2_ref[...]

    def body(k, carry):
        s, q = carry
        y1 = y1_ref[0, pl.ds(k, 1), :, :][0]
        z1 = _bf16(_lrelu(y1 * a1 + b1))
        y2 = jnp.dot(W2, z1, preferred_element_type=F32)
        y2_ref[0, pl.ds(k, 1), :, :] = y2[None]
        return (s + jnp.sum(y2, axis=1, keepdims=True),
                q + jnp.sum(y2 * y2, axis=1, keepdims=True))

    s, q = jax.lax.fori_loop(0, K, body,
                             (jnp.zeros((64, 1), F32), jnp.zeros((64, 1), F32)),
                             unroll=False)
    st = jnp.concatenate([s, q], axis=1)

    @pl.when((b == 0) & (t == 0))
    def _():
        st_ref[...] = st

    @pl.when((b != 0) | (t != 0))
    def _():
        st_ref[...] += st


def _bn_conv2(y1buf, a1, b1, w2):
    return pl.pallas_call(
        _bn_conv2_kernel,
        out_shape=(
            jax.ShapeDtypeStruct((B, K, 64, N), F32),
            jax.ShapeDtypeStruct((64, 2), F32),
        ),
        grid=(B, NT),
        in_specs=[
            pl.BlockSpec((1, K, 64, T), lambda b, t: (b, 0, 0, t)),
            pl.BlockSpec((64, 1), lambda b, t: (0, 0)),
            pl.BlockSpec((64, 1), lambda b, t: (0, 0)),
            pl.BlockSpec(w2.shape, lambda b, t: (0, 0)),
        ],
        out_specs=(
            pl.BlockSpec((1, K, 64, T), lambda b, t: (b, 0, 0, t)),
            pl.BlockSpec((64, 2), lambda b, t: (0, 0)),
        ),
    )(y1buf, a1, b1, w2)


def _maxpool_kernel(y2_ref, a2_ref, b2_ref, x1_ref):
    a2 = a2_ref[...]
    b2 = b2_ref[...]

    def body(k, m):
        y2 = y2_ref[0, pl.ds(k, 1), :, :][0]
        return jnp.maximum(m, _lrelu(y2 * a2 + b2))

    x1_ref[0] = jax.lax.fori_loop(0, K, body,
                                  jnp.full((64, T), NEG, F32), unroll=False)


def _edge_maxpool(y2buf, a2, b2):
    return pl.pallas_call(
        _maxpool_kernel,
        out_shape=jax.ShapeDtypeStruct((B, 64, N), F32),
        grid=(B, NT),
        in_specs=[
            pl.BlockSpec((1, K, 64, T), lambda b, t: (b, 0, 0, t)),
            pl.BlockSpec((64, 1), lambda b, t: (0, 0)),
            pl.BlockSpec((64, 1), lambda b, t: (0, 0)),
        ],
        out_specs=pl.BlockSpec((1, 64, T), lambda b, t: (b, 0, t)),
    )(y2buf, a2, b2)


def _conv6_kernel(x1r, x2r, x3r, wr, yr, st_ref):
    b = pl.program_id(0)
    t = pl.program_id(1)
    xcat = _bf16(jnp.concatenate([x1r[0], x2r[0], x3r[0]], axis=0))
    y = jnp.dot(wr[...], xcat, preferred_element_type=F32)
    yr[0] = y
    st = jnp.concatenate([jnp.sum(y, axis=1, keepdims=True),
                          jnp.sum(y * y, axis=1, keepdims=True)], axis=1)

    @pl.when((b == 0) & (t == 0))
    def _():
        st_ref[...] = st

    @pl.when((b != 0) | (t != 0))
    def _():
        st_ref[...] += st


def _conv6(x1, x2, x3, w6):
    return pl.pallas_call(
        _conv6_kernel,
        out_shape=(
            jax.ShapeDtypeStruct((B, 1024, N), F32),
            jax.ShapeDtypeStruct((1024, 2), F32),
        ),
        grid=(B, NT),
        in_specs=[
            pl.BlockSpec((1, 64, T), lambda b, t: (b, 0, t)),
            pl.BlockSpec((1, 64, T), lambda b, t: (b, 0, t)),
            pl.BlockSpec((1, 64, T), lambda b, t: (b, 0, t)),
            pl.BlockSpec((1024, 192), lambda b, t: (0, 0)),
        ],
        out_specs=(
            pl.BlockSpec((1, 1024, T), lambda b, t: (b, 0, t)),
            pl.BlockSpec((1024, 2), lambda b, t: (0, 0)),
        ),
    )(x1, x2, x3, w6)


def _globalpool_kernel(y6_ref, a6_ref, b6_ref, g_ref):
    z = _lrelu(y6_ref[0] * a6_ref[...] + b6_ref[...])
    g_ref[0] = jnp.max(z, axis=1, keepdims=True)


def _globalpool(y6, a6, b6):
    return pl.pallas_call(
        _globalpool_kernel,
        out_shape=jax.ShapeDtypeStruct((B, 1024, 1), F32),
        grid=(B,),
        in_specs=[
            pl.BlockSpec((1, 1024, N), lambda b: (b, 0, 0)),
            pl.BlockSpec((1024, 1), lambda b: (0, 0)),
            pl.BlockSpec((1024, 1), lambda b: (0, 0)),
        ],
        out_specs=pl.BlockSpec((1, 1024, 1), lambda b: (b, 0, 0)),
    )(y6, a6, b6)


def _conv7_kernel(g_ref, x1_ref, x2_ref, x3_ref, wg_ref, wl_ref, y_ref,
                  st_ref):
    b = pl.program_id(0)
    t = pl.program_id(1)
    base = jnp.dot(wg_ref[...], _bf16(g_ref[0]),
                   preferred_element_type=F32)
    xcat = _bf16(jnp.concatenate([x1_ref[0], x2_ref[0], x3_ref[0]], axis=0))
    y = base + jnp.dot(wl_ref[...], xcat, preferred_element_type=F32)
    y_ref[0] = y
    st = jnp.concatenate([jnp.sum(y, axis=1, keepdims=True),
                          jnp.sum(y * y, axis=1, keepdims=True)], axis=1)

    @pl.when((b == 0) & (t == 0))
    def _():
        st_ref[...] = st

    @pl.when((b != 0) | (t != 0))
    def _():
        st_ref[...] += st


def _conv7(g, x1, x2, x3, w7g, w7l):
    return pl.pallas_call(
        _conv7_kernel,
        out_shape=(
            jax.ShapeDtypeStruct((B, 512, N), F32),
            jax.ShapeDtypeStruct((512, 2), F32),
        ),
        grid=(B, NT),
        in_specs=[
            pl.BlockSpec((1, 1024, 1), lambda b, t: (b, 0, 0)),
            pl.BlockSpec((1, 64, T), lambda b, t: (b, 0, t)),
            pl.BlockSpec((1, 64, T), lambda b, t: (b, 0, t)),
            pl.BlockSpec((1, 64, T), lambda b, t: (b, 0, t)),
            pl.BlockSpec((512, 1024), lambda b, t: (0, 0)),
            pl.BlockSpec((512, 192), lambda b, t: (0, 0)),
        ],
        out_specs=(
            pl.BlockSpec((1, 512, T), lambda b, t: (b, 0, t)),
            pl.BlockSpec((512, 2), lambda b, t: (0, 0)),
        ),
    )(g, x1, x2, x3, w7g, w7l)


def _bn_conv_kernel(y_ref, a_ref, b_ref, w_ref, yo_ref, st_ref):
    b = pl.program_id(0)
    t = pl.program_id(1)
    z = _bf16(_lrelu(y_ref[0] * a_ref[...] + b_ref[...]))
    y = jnp.dot(w_ref[...], z, preferred_element_type=F32)
    yo_ref[0] = y
    st = jnp.concatenate([jnp.sum(y, axis=1, keepdims=True),
                          jnp.sum(y * y, axis=1, keepdims=True)], axis=1)

    @pl.when((b == 0) & (t == 0))
    def _():
        st_ref[...] = st

    @pl.when((b != 0) | (t != 0))
    def _():
        st_ref[...] += st


def _bn_conv(y, a, bb, w):
    cin = y.shape[1]
    cout = w.shape[0]
    return pl.pallas_call(
        _bn_conv_kernel,
        out_shape=(
            jax.ShapeDtypeStruct((B, cout, N), F32),
            jax.ShapeDtypeStruct((cout, 2), F32),
        ),
        grid=(B, NT),
        in_specs=[
            pl.BlockSpec((1, cin, T), lambda b, t: (b, 0, t)),
            pl.BlockSpec((cin, 1), lambda b, t: (0, 0)),
            pl.BlockSpec((cin, 1), lambda b, t: (0, 0)),
            pl.BlockSpec((cout, cin), lambda b, t: (0, 0)),
        ],
        out_specs=(
            pl.BlockSpec((1, cout, T), lambda b, t: (b, 0, t)),
            pl.BlockSpec((cout, 2), lambda b, t: (0, 0)),
        ),
    )(y, a, bb, w)


def _final_kernel(y_ref, a_ref, b_ref, w_ref, b9_ref, o_ref):
    z = _bf16(_lrelu(y_ref[0] * a_ref[...] + b_ref[...]))
    o_ref[0] = jnp.dot(w_ref[...], z, preferred_element_type=F32) + b9_ref[...]


def _final(y8, a8, b8, w9, b9):
    return pl.pallas_call(
        _final_kernel,
        out_shape=jax.ShapeDtypeStruct((B, 15, N), F32),
        grid=(B, NT),
        in_specs=[
            pl.BlockSpec((1, 256, T), lambda b, t: (b, 0, t)),
            pl.BlockSpec((256, 1), lambda b, t: (0, 0)),
            pl.BlockSpec((256, 1), lambda b, t: (0, 0)),
            pl.BlockSpec((15, 256), lambda b, t: (0, 0)),
            pl.BlockSpec((15, 1), lambda b, t: (0, 0)),
        ],
        out_specs=pl.BlockSpec((1, 15, T), lambda b, t: (b, 0, t)),
    )(y8, a8, b8, w9, b9)


def kernel(x_s, W1, W2, W3, W4, W5, W6, W7, W8, W9, b9,
           g1, bt1, g2, bt2, g3, bt3, g4, bt4, g5, bt5,
           g6, bt6, g7, bt7, g8, bt8):
    cnt2d = float(B * N * K)
    cnt1d = float(B * N)
    W1r, W2r, W3r, W4r, W5r, W6r, W7r, W8r, W9r = (
        _bf16(W) for W in (W1, W2, W3, W4, W5, W6, W7, W8, W9))

    idx = _knn(x_s)

    y1buf, st1 = _edge_conv1(x_s, idx, W1r)
    a1, b1 = _bn_coef(st1, g1, bt1, cnt2d)
    y2buf, st2 = _bn_conv2(y1buf, a1, b1, W2r)
    a2, b2 = _bn_coef(st2, g2, bt2, cnt2d)
    x1 = _edge_maxpool(y2buf, a2, b2)

    y3buf, st3 = _edge_conv1(x1, idx, W3r)
    a3, b3 = _bn_coef(st3, g3, bt3, cnt2d)
    y4buf, st4 = _bn_conv2(y3buf, a3, b3, W4r)
    a4, b4 = _bn_coef(st4, g4, bt4, cnt2d)
    x2 = _edge_maxpool(y4buf, a4, b4)

    y5buf, st5 = _edge_conv1(x2, idx, W5r)
    a5, b5 = _bn_coef(st5, g5, bt5, cnt2d)
    x3 = _edge_maxpool(y5buf, a5, b5)

    y6, st6 = _conv6(x1, x2, x3, W6r)
    a6, b6 = _bn_coef(st6, g6, bt6, cnt1d)
    g = _globalpool(y6, a6, b6)
    y7, st7 = _conv7(g, x1, x2, x3, W7r[:, :1024], W7r[:, 1024:])
    a7, b7 = _bn_coef(st7, g7, bt7, cnt1d)
    y8, st8 = _bn_conv(y7, a7, b7, W8r)
    a8, b8 = _bn_coef(st8, g8, bt8, cnt1d)
    out = _final(y8, a8, b8, W9r, b9[:, None])
    return (out, jnp.array([0.0], dtype=F32))


# speedup vs baseline: 1.6320x; 1.6320x over previous
import jax
import jax.numpy as jnp
from jax.experimental import pallas as pl
from jax.experimental.pallas import tpu as pltpu

B, N, K = 8, 1024, 40
EPS = 1e-5
T = 256
NT = N // T
NEG = -jnp.inf

F32 = jnp.float32
HI = jax.lax.Precision.HIGHEST


def _lrelu(x):
    return jnp.where(x >= 0, x, 0.2 * x)


def _bf16(x):
    return x.astype(jnp.bfloat16).astype(F32)


def _knn_kernel(x_ref, idx_ref, d_ref):
    x = x_ref[0]
    xb = _bf16(x)
    xbT = xb.T
    G = xbT[:, 0:1] * xb[0:1, :]
    G = G + xbT[:, 1:2] * xb[1:2, :]
    G = G + xbT[:, 2:3] * xb[2:3, :]
    inner = -2.0 * G
    x2 = x * x
    xx_lane = jnp.sum(x2, axis=0, keepdims=True)
    xT = x.T
    xx_sub = jnp.sum(xT * xT, axis=1, keepdims=True)
    d_ref[...] = ((0.0 - xx_sub) - inner) - xx_lane
    iota = jax.lax.broadcasted_iota(jnp.int32, (N, N), 0)

    def body(k, _):
        D = d_ref[...]
        m = jnp.max(D, axis=0, keepdims=True)
        sel = jnp.min(jnp.where(D == m, iota, N), axis=0, keepdims=True)
        idx_ref[0, pl.ds(k, 1), :] = sel
        d_ref[...] = jnp.where(iota == sel, NEG, D)
        return 0

    jax.lax.fori_loop(0, K, body, 0, unroll=False)


def _knn(x_s):
    return pl.pallas_call(
        _knn_kernel,
        out_shape=jax.ShapeDtypeStruct((B, K, N), jnp.int32),
        grid=(B,),
        in_specs=[pl.BlockSpec((1, 3, N), lambda b: (b, 0, 0))],
        out_specs=pl.BlockSpec((1, K, N), lambda b: (b, 0, 0)),
        scratch_shapes=[pltpu.VMEM((N, N), F32)],
    )(x_s)


def _bn_coef(st, g, bt, cnt):
    mean = st[:, 0] / cnt
    var = st[:, 1] / cnt - mean * mean
    a = g / jnp.sqrt(var + EPS)
    b = bt - mean * a
    return a[:, None], b[:, None]


def _edge_conv1_kernel(x_ref, xt_ref, idx_ref, w1_ref, y1_ref, st_ref):
    b = pl.program_id(0)
    t = pl.program_id(1)
    x = x_ref[0]
    xt = xt_ref[0]
    W1 = w1_ref[...]
    cin = x.shape[0]

    def body(k, carry):
        s, q = carry
        ik = idx_ref[0, pl.ds(k, 1), :]
        xg = jnp.zeros((cin, T), F32)
        for g in range(N // 128):
            rel = ik - (g * 128)
            inb = (rel >= 0) & (rel < 128)
            idxb = jnp.broadcast_to(jnp.where(inb, rel, 0), (cin, T))
            gat = jnp.take_along_axis(x[:, g * 128:(g + 1) * 128],
                                      idxb, axis=1)
            xg = jnp.where(inb, gat, xg)
        ef = _bf16(jnp.concatenate([xg - xt, xt], axis=0))
        y1 = jnp.dot(W1, ef, preferred_element_type=F32)
        y1_ref[0, pl.ds(k, 1), :, :] = y1[None]
        return (s + jnp.sum(y1, axis=1, keepdims=True),
                q + jnp.sum(y1 * y1, axis=1, keepdims=True))

    s, q = jax.lax.fori_loop(0, K, body,
                             (jnp.zeros((64, 1), F32), jnp.zeros((64, 1), F32)),
                             unroll=False)
    st = jnp.concatenate([s, q], axis=1)

    @pl.when((b == 0) & (t == 0))
    def _():
        st_ref[...] = st

    @pl.when((b != 0) | (t != 0))
    def _():
        st_ref[...] += st


def _edge_conv1(x, idx, w1):
    cin = x.shape[1]
    return pl.pallas_call(
        _edge_conv1_kernel,
        out_shape=(
            jax.ShapeDtypeStruct((B, K, 64, N), F32),
            jax.ShapeDtypeStruct((64, 2), F32),
        ),
        grid=(B, NT),
        in_specs=[
            pl.BlockSpec((1, cin, N), lambda b, t: (b, 0, 0)),
            pl.BlockSpec((1, cin, T), lambda b, t: (b, 0, t)),
            pl.BlockSpec((1, K, T), lambda b, t: (b, 0, t)),
            pl.BlockSpec(w1.shape, lambda b, t: (0, 0)),
        ],
        out_specs=(
            pl.BlockSpec((1, K, 64, T), lambda b, t: (b, 0, 0, t)),
            pl.BlockSpec((64, 2), lambda b, t: (0, 0)),
        ),
    )(x, x, idx, w1)


def _bn_conv2_kernel(y1_ref, a1_ref, b1_ref, w2_ref, y2_ref, st_ref):
    b = pl.program_id(0)
    t = pl.program_id(1)
    a1 = a1_ref[...]
    b1 = b1_ref[...]
    W2 = w2_ref[...]

    def body(k, carry):
        s, q = carry
        y1 = y1_ref[0, pl.ds(k, 1), :, :][0]
        z1 = _bf16(_lrelu(y1 * a1 + b1))
        y2 = jnp.dot(W2, z1, preferred_element_type=F32)
        y2_ref[0, pl.ds(k, 1), :, :] = y2[None]
        return (s + jnp.sum(y2, axis=1, keepdims=True),
                q + jnp.sum(y2 * y2, axis=1, keepdims=True))

    s, q = jax.lax.fori_loop(0, K, body,
                             (jnp.zeros((64, 1), F32), jnp.zeros((64, 1), F32)),
                             unroll=False)
    st = jnp.concatenate([s, q], axis=1)

    @pl.when((b == 0) & (t == 0))
    def _():
        st_ref[...] = st

    @pl.when((b != 0) | (t != 0))
    def _():
        st_ref[...] += st


def _bn_conv2(y1buf, a1, b1, w2):
    return pl.pallas_call(
        _bn_conv2_kernel,
        out_shape=(
            jax.ShapeDtypeStruct((B, K, 64, N), F32),
            jax.ShapeDtypeStruct((64, 2), F32),
        ),
        grid=(B, NT),
        in_specs=[
            pl.BlockSpec((1, K, 64, T), lambda b, t: (b, 0, 0, t)),
            pl.BlockSpec((64, 1), lambda b, t: (0, 0)),
            pl.BlockSpec((64, 1), lambda b, t: (0, 0)),
            pl.BlockSpec(w2.shape, lambda b, t: (0, 0)),
        ],
        out_specs=(
            pl.BlockSpec((1, K, 64, T), lambda b, t: (b, 0, 0, t)),
            pl.BlockSpec((64, 2), lambda b, t: (0, 0)),
        ),
    )(y1buf, a1, b1, w2)


def _maxpool_kernel(y2_ref, a2_ref, b2_ref, x1_ref):
    a2 = a2_ref[...]
    b2 = b2_ref[...]

    def body(k, m):
        y2 = y2_ref[0, pl.ds(k, 1), :, :][0]
        return jnp.maximum(m, _lrelu(y2 * a2 + b2))

    x1_ref[0] = jax.lax.fori_loop(0, K, body,
                                  jnp.full((64, T), NEG, F32), unroll=False)


def _edge_maxpool(y2buf, a2, b2):
    return pl.pallas_call(
        _maxpool_kernel,
        out_shape=jax.ShapeDtypeStruct((B, 64, N), F32),
        grid=(B, NT),
        in_specs=[
            pl.BlockSpec((1, K, 64, T), lambda b, t: (b, 0, 0, t)),
            pl.BlockSpec((64, 1), lambda b, t: (0, 0)),
            pl.BlockSpec((64, 1), lambda b, t: (0, 0)),
        ],
        out_specs=pl.BlockSpec((1, 64, T), lambda b, t: (b, 0, t)),
    )(y2buf, a2, b2)


def _conv6_kernel(x1r, x2r, x3r, wr, yr, st_ref):
    b = pl.program_id(0)
    t = pl.program_id(1)
    xcat = _bf16(jnp.concatenate([x1r[0], x2r[0], x3r[0]], axis=0))
    y = jnp.dot(wr[...], xcat, preferred_element_type=F32)
    yr[0] = y
    st = jnp.concatenate([jnp.sum(y, axis=1, keepdims=True),
                          jnp.sum(y * y, axis=1, keepdims=True)], axis=1)

    @pl.when((b == 0) & (t == 0))
    def _():
        st_ref[...] = st

    @pl.when((b != 0) | (t != 0))
    def _():
        st_ref[...] += st


def _conv6(x1, x2, x3, w6):
    return pl.pallas_call(
        _conv6_kernel,
        out_shape=(
            jax.ShapeDtypeStruct((B, 1024, N), F32),
            jax.ShapeDtypeStruct((1024, 2), F32),
        ),
        grid=(B, NT),
        in_specs=[
            pl.BlockSpec((1, 64, T), lambda b, t: (b, 0, t)),
            pl.BlockSpec((1, 64, T), lambda b, t: (b, 0, t)),
            pl.BlockSpec((1, 64, T), lambda b, t: (b, 0, t)),
            pl.BlockSpec((1024, 192), lambda b, t: (0, 0)),
        ],
        out_specs=(
            pl.BlockSpec((1, 1024, T), lambda b, t: (b, 0, t)),
            pl.BlockSpec((1024, 2), lambda b, t: (0, 0)),
        ),
    )(x1, x2, x3, w6)


def _globalpool_kernel(y6_ref, a6_ref, b6_ref, g_ref):
    z = _lrelu(y6_ref[0] * a6_ref[...] + b6_ref[...])
    g_ref[0] = jnp.max(z, axis=1, keepdims=True)


def _globalpool(y6, a6, b6):
    return pl.pallas_call(
        _globalpool_kernel,
        out_shape=jax.ShapeDtypeStruct((B, 1024, 1), F32),
        grid=(B,),
        in_specs=[
            pl.BlockSpec((1, 1024, N), lambda b: (b, 0, 0)),
            pl.BlockSpec((1024, 1), lambda b: (0, 0)),
            pl.BlockSpec((1024, 1), lambda b: (0, 0)),
        ],
        out_specs=pl.BlockSpec((1, 1024, 1), lambda b: (b, 0, 0)),
    )(y6, a6, b6)


def _conv7_kernel(g_ref, x1_ref, x2_ref, x3_ref, wg_ref, wl_ref, y_ref,
                  st_ref):
    b = pl.program_id(0)
    t = pl.program_id(1)
    base = jnp.dot(wg_ref[...], _bf16(g_ref[0]),
                   preferred_element_type=F32)
    xcat = _bf16(jnp.concatenate([x1_ref[0], x2_ref[0], x3_ref[0]], axis=0))
    y = base + jnp.dot(wl_ref[...], xcat, preferred_element_type=F32)
    y_ref[0] = y
    st = jnp.concatenate([jnp.sum(y, axis=1, keepdims=True),
                          jnp.sum(y * y, axis=1, keepdims=True)], axis=1)

    @pl.when((b == 0) & (t == 0))
    def _():
        st_ref[...] = st

    @pl.when((b != 0) | (t != 0))
    def _():
        st_ref[...] += st


def _conv7(g, x1, x2, x3, w7g, w7l):
    return pl.pallas_call(
        _conv7_kernel,
        out_shape=(
            jax.ShapeDtypeStruct((B, 512, N), F32),
            jax.ShapeDtypeStruct((512, 2), F32),
        ),
        grid=(B, NT),
        in_specs=[
            pl.BlockSpec((1, 1024, 1), lambda b, t: (b, 0, 0)),
            pl.BlockSpec((1, 64, T), lambda b, t: (b, 0, t)),
            pl.BlockSpec((1, 64, T), lambda b, t: (b, 0, t)),
            pl.BlockSpec((1, 64, T), lambda b, t: (b, 0, t)),
            pl.BlockSpec((512, 1024), lambda b, t: (0, 0)),
            pl.BlockSpec((512, 192), lambda b, t: (0, 0)),
        ],
        out_specs=(
            pl.BlockSpec((1, 512, T), lambda b, t: (b, 0, t)),
            pl.BlockSpec((512, 2), lambda b, t: (0, 0)),
        ),
    )(g, x1, x2, x3, w7g, w7l)


def _bn_conv_kernel(y_ref, a_ref, b_ref, w_ref, yo_ref, st_ref):
    b = pl.program_id(0)
    t = pl.program_id(1)
    z = _bf16(_lrelu(y_ref[0] * a_ref[...] + b_ref[...]))
    y = jnp.dot(w_ref[...], z, preferred_element_type=F32)
    yo_ref[0] = y
    st = jnp.concatenate([jnp.sum(y, axis=1, keepdims=True),
                          jnp.sum(y * y, axis=1, keepdims=True)], axis=1)

    @pl.when((b == 0) & (t == 0))
    def _():
        st_ref[...] = st

    @pl.when((b != 0) | (t != 0))
    def _():
        st_ref[...] += st


def _bn_conv(y, a, bb, w):
    cin = y.shape[1]
    cout = w.shape[0]
    return pl.pallas_call(
        _bn_conv_kernel,
        out_shape=(
            jax.ShapeDtypeStruct((B, cout, N), F32),
            jax.ShapeDtypeStruct((cout, 2), F32),
        ),
        grid=(B, NT),
        in_specs=[
            pl.BlockSpec((1, cin, T), lambda b, t: (b, 0, t)),
            pl.BlockSpec((cin, 1), lambda b, t: (0, 0)),
            pl.BlockSpec((cin, 1), lambda b, t: (0, 0)),
            pl.BlockSpec((cout, cin), lambda b, t: (0, 0)),
        ],
        out_specs=(
            pl.BlockSpec((1, cout, T), lambda b, t: (b, 0, t)),
            pl.BlockSpec((cout, 2), lambda b, t: (0, 0)),
        ),
    )(y, a, bb, w)


def _final_kernel(y_ref, a_ref, b_ref, w_ref, b9_ref, o_ref):
    z = _bf16(_lrelu(y_ref[0] * a_ref[...] + b_ref[...]))
    o_ref[0] = jnp.dot(w_ref[...], z, preferred_element_type=F32) + b9_ref[...]


def _final(y8, a8, b8, w9, b9):
    return pl.pallas_call(
        _final_kernel,
        out_shape=jax.ShapeDtypeStruct((B, 15, N), F32),
        grid=(B, NT),
        in_specs=[
            pl.BlockSpec((1, 256, T), lambda b, t: (b, 0, t)),
            pl.BlockSpec((256, 1), lambda b, t: (0, 0)),
            pl.BlockSpec((256, 1), lambda b, t: (0, 0)),
            pl.BlockSpec((15, 256), lambda b, t: (0, 0)),
            pl.BlockSpec((15, 1), lambda b, t: (0, 0)),
        ],
        out_specs=pl.BlockSpec((1, 15, T), lambda b, t: (b, 0, t)),
    )(y8, a8, b8, w9, b9)


def kernel(x_s, W1, W2, W3, W4, W5, W6, W7, W8, W9, b9,
           g1, bt1, g2, bt2, g3, bt3, g4, bt4, g5, bt5,
           g6, bt6, g7, bt7, g8, bt8):
    cnt2d = float(B * N * K)
    cnt1d = float(B * N)
    W1r, W2r, W3r, W4r, W5r, W6r, W7r, W8r, W9r = (
        _bf16(W) for W in (W1, W2, W3, W4, W5, W6, W7, W8, W9))

    idx = _knn(x_s)

    y1buf, st1 = _edge_conv1(x_s, idx, W1r)
    a1, b1 = _bn_coef(st1, g1, bt1, cnt2d)
    y2buf, st2 = _bn_conv2(y1buf, a1, b1, W2r)
    a2, b2 = _bn_coef(st2, g2, bt2, cnt2d)
    x1 = _edge_maxpool(y2buf, a2, b2)

    y3buf, st3 = _edge_conv1(x1, idx, W3r)
    a3, b3 = _bn_coef(st3, g3, bt3, cnt2d)
    y4buf, st4 = _bn_conv2(y3buf, a3, b3, W4r)
    a4, b4 = _bn_coef(st4, g4, bt4, cnt2d)
    x2 = _edge_maxpool(y4buf, a4, b4)

    y5buf, st5 = _edge_conv1(x2, idx, W5r)
    a5, b5 = _bn_coef(st5, g5, bt5, cnt2d)
    x3 = _edge_maxpool(y5buf, a5, b5)

    y6, st6 = _conv6(x1, x2, x3, W6r)
    a6, b6 = _bn_coef(st6, g6, bt6, cnt1d)
    g = _globalpool(y6, a6, b6)
    y7, st7 = _conv7(g, x1, x2, x3, W7r[:, :1024], W7r[:, 1024:])
    a7, b7 = _bn_coef(st7, g7, bt7, cnt1d)
    y8, st8 = _bn_conv(y7, a7, b7, W8r)
    a8, b8 = _bn_coef(st8, g8, bt8, cnt1d)
    out = _final(y8, a8, b8, W9r, b9[:, None])
    return (out, jnp.array([0.0], dtype=F32))


# speedup vs baseline: 2.2303x; 1.3666x over previous
import jax
import jax.numpy as jnp
from jax.experimental import pallas as pl
from jax.experimental.pallas import tpu as pltpu

B, N, K = 8, 1024, 40
EPS = 1e-5
T = 512
NT = N // T
NEG = -jnp.inf

F32 = jnp.float32
HI = jax.lax.Precision.HIGHEST


def _lrelu(x):
    return jnp.where(x >= 0, x, 0.2 * x)


def _bf16(x):
    return x.astype(jnp.bfloat16).astype(F32)


def _knn_kernel(x_ref, idx_ref, d_ref):
    x = x_ref[0]
    xb = _bf16(x)
    xbT = xb.T
    G = xbT[:, 0:1] * xb[0:1, :]
    G = G + xbT[:, 1:2] * xb[1:2, :]
    G = G + xbT[:, 2:3] * xb[2:3, :]
    inner = -2.0 * G
    x2 = x * x
    xx_lane = jnp.sum(x2, axis=0, keepdims=True)
    xT = x.T
    xx_sub = jnp.sum(xT * xT, axis=1, keepdims=True)
    d_ref[...] = ((0.0 - xx_sub) - inner) - xx_lane
    iota = jax.lax.broadcasted_iota(jnp.int32, (N, N), 0)

    def body(k, _):
        D = d_ref[...]
        m = jnp.max(D, axis=0, keepdims=True)
        sel = jnp.min(jnp.where(D == m, iota, N), axis=0, keepdims=True)
        idx_ref[0, pl.ds(k, 1), :] = sel
        d_ref[...] = jnp.where(iota == sel, NEG, D)
        return 0

    jax.lax.fori_loop(0, K, body, 0, unroll=False)


def _knn(x_s):
    return pl.pallas_call(
        _knn_kernel,
        out_shape=jax.ShapeDtypeStruct((B, K, N), jnp.int32),
        grid=(B,),
        in_specs=[pl.BlockSpec((1, 3, N), lambda b: (b, 0, 0))],
        out_specs=pl.BlockSpec((1, K, N), lambda b: (b, 0, 0)),
        scratch_shapes=[pltpu.VMEM((N, N), F32)],
    )(x_s)


def _bn_coef(st, g, bt, cnt):
    mean = st[:, 0] / cnt
    var = st[:, 1] / cnt - mean * mean
    a = g / jnp.sqrt(var + EPS)
    b = bt - mean * a
    return a[:, None], b[:, None]


def _edge_conv1_kernel(x_ref, xt_ref, idx_ref, w1_ref, y1_ref, st_ref):
    b = pl.program_id(0)
    t = pl.program_id(1)
    x = x_ref[0]
    xt = xt_ref[0]
    W1 = w1_ref[...]
    cin = x.shape[0]

    def body(k, carry):
        s, q = carry
        ik = idx_ref[0, pl.ds(k, 1), :]
        xg = jnp.zeros((cin, T), F32)
        for g in range(N // 128):
            rel = ik - (g * 128)
            inb = (rel >= 0) & (rel < 128)
            idxb = jnp.broadcast_to(jnp.where(inb, rel, 0), (cin, T))
            gat = jnp.take_along_axis(x[:, g * 128:(g + 1) * 128],
                                      idxb, axis=1)
            xg = jnp.where(inb, gat, xg)
        ef = _bf16(jnp.concatenate([xg - xt, xt], axis=0))
        y1 = jnp.dot(W1, ef, preferred_element_type=F32)
        y1_ref[0, pl.ds(k, 1), :, :] = y1[None]
        return (s + jnp.sum(y1, axis=1, keepdims=True),
                q + jnp.sum(y1 * y1, axis=1, keepdims=True))

    s, q = jax.lax.fori_loop(0, K, body,
                             (jnp.zeros((64, 1), F32), jnp.zeros((64, 1), F32)),
                             unroll=False)
    st = jnp.concatenate([s, q], axis=1)

    @pl.when((b == 0) & (t == 0))
    def _():
        st_ref[...] = st

    @pl.when((b != 0) | (t != 0))
    def _():
        st_ref[...] += st


def _edge_conv1(x, idx, w1):
    cin = x.shape[1]
    return pl.pallas_call(
        _edge_conv1_kernel,
        out_shape=(
            jax.ShapeDtypeStruct((B, K, 64, N), F32),
            jax.ShapeDtypeStruct((64, 2), F32),
        ),
        grid=(B, NT),
        in_specs=[
            pl.BlockSpec((1, cin, N), lambda b, t: (b, 0, 0)),
            pl.BlockSpec((1, cin, T), lambda b, t: (b, 0, t)),
            pl.BlockSpec((1, K, T), lambda b, t: (b, 0, t)),
            pl.BlockSpec(w1.shape, lambda b, t: (0, 0)),
        ],
        out_specs=(
            pl.BlockSpec((1, K, 64, T), lambda b, t: (b, 0, 0, t)),
            pl.BlockSpec((64, 2), lambda b, t: (0, 0)),
        ),
    )(x, x, idx, w1)


def _bn_conv2_kernel(y1_ref, a1_ref, b1_ref, w2_ref, y2_ref, st_ref):
    b = pl.program_id(0)
    t = pl.program_id(1)
    a1 = a1_ref[...]
    b1 = b1_ref[...]
    W2 = w2_ref[...]

    def body(k, carry):
        s, q = carry
        y1 = y1_ref[0, pl.ds(k, 1), :, :][0]
        z1 = _bf16(_lrelu(y1 * a1 + b1))
        y2 = jnp.dot(W2, z1, preferred_element_type=F32)
        y2_ref[0, pl.ds(k, 1), :, :] = y2[None]
        return (s + jnp.sum(y2, axis=1, keepdims=True),
                q + jnp.sum(y2 * y2, axis=1, keepdims=True))

    s, q = jax.lax.fori_loop(0, K, body,
                             (jnp.zeros((64, 1), F32), jnp.zeros((64, 1), F32)),
                             unroll=False)
    st = jnp.concatenate([s, q], axis=1)

    @pl.when((b == 0) & (t == 0))
    def _():
        st_ref[...] = st

    @pl.when((b != 0) | (t != 0))
    def _():
        st_ref[...] += st


def _bn_conv2(y1buf, a1, b1, w2):
    return pl.pallas_call(
        _bn_conv2_kernel,
        out_shape=(
            jax.ShapeDtypeStruct((B, K, 64, N), F32),
            jax.ShapeDtypeStruct((64, 2), F32),
        ),
        grid=(B, NT),
        in_specs=[
            pl.BlockSpec((1, K, 64, T), lambda b, t: (b, 0, 0, t)),
            pl.BlockSpec((64, 1), lambda b, t: (0, 0)),
            pl.BlockSpec((64, 1), lambda b, t: (0, 0)),
            pl.BlockSpec(w2.shape, lambda b, t: (0, 0)),
        ],
        out_specs=(
            pl.BlockSpec((1, K, 64, T), lambda b, t: (b, 0, 0, t)),
            pl.BlockSpec((64, 2), lambda b, t: (0, 0)),
        ),
    )(y1buf, a1, b1, w2)


def _maxpool_kernel(y2_ref, a2_ref, b2_ref, x1_ref):
    a2 = a2_ref[...]
    b2 = b2_ref[...]

    def body(k, m):
        y2 = y2_ref[0, pl.ds(k, 1), :, :][0]
        return jnp.maximum(m, _lrelu(y2 * a2 + b2))

    x1_ref[0] = jax.lax.fori_loop(0, K, body,
                                  jnp.full((64, T), NEG, F32), unroll=False)


def _edge_maxpool(y2buf, a2, b2):
    return pl.pallas_call(
        _maxpool_kernel,
        out_shape=jax.ShapeDtypeStruct((B, 64, N), F32),
        grid=(B, NT),
        in_specs=[
            pl.BlockSpec((1, K, 64, T), lambda b, t: (b, 0, 0, t)),
            pl.BlockSpec((64, 1), lambda b, t: (0, 0)),
            pl.BlockSpec((64, 1), lambda b, t: (0, 0)),
        ],
        out_specs=pl.BlockSpec((1, 64, T), lambda b, t: (b, 0, t)),
    )(y2buf, a2, b2)


def _conv6_kernel(x1r, x2r, x3r, wr, yr, st_ref):
    b = pl.program_id(0)
    t = pl.program_id(1)
    xcat = _bf16(jnp.concatenate([x1r[0], x2r[0], x3r[0]], axis=0))
    y = jnp.dot(wr[...], xcat, preferred_element_type=F32)
    yr[0] = y
    st = jnp.concatenate([jnp.sum(y, axis=1, keepdims=True),
                          jnp.sum(y * y, axis=1, keepdims=True)], axis=1)

    @pl.when((b == 0) & (t == 0))
    def _():
        st_ref[...] = st

    @pl.when((b != 0) | (t != 0))
    def _():
        st_ref[...] += st


def _conv6(x1, x2, x3, w6):
    return pl.pallas_call(
        _conv6_kernel,
        out_shape=(
            jax.ShapeDtypeStruct((B, 1024, N), F32),
            jax.ShapeDtypeStruct((1024, 2), F32),
        ),
        grid=(B, NT),
        in_specs=[
            pl.BlockSpec((1, 64, T), lambda b, t: (b, 0, t)),
            pl.BlockSpec((1, 64, T), lambda b, t: (b, 0, t)),
            pl.BlockSpec((1, 64, T), lambda b, t: (b, 0, t)),
            pl.BlockSpec((1024, 192), lambda b, t: (0, 0)),
        ],
        out_specs=(
            pl.BlockSpec((1, 1024, T), lambda b, t: (b, 0, t)),
            pl.BlockSpec((1024, 2), lambda b, t: (0, 0)),
        ),
    )(x1, x2, x3, w6)


def _globalpool_kernel(y6_ref, a6_ref, b6_ref, g_ref):
    z = _lrelu(y6_ref[0] * a6_ref[...] + b6_ref[...])
    g_ref[0] = jnp.max(z, axis=1, keepdims=True)


def _globalpool(y6, a6, b6):
    return pl.pallas_call(
        _globalpool_kernel,
        out_shape=jax.ShapeDtypeStruct((B, 1024, 1), F32),
        grid=(B,),
        in_specs=[
            pl.BlockSpec((1, 1024, N), lambda b: (b, 0, 0)),
            pl.BlockSpec((1024, 1), lambda b: (0, 0)),
            pl.BlockSpec((1024, 1), lambda b: (0, 0)),
        ],
        out_specs=pl.BlockSpec((1, 1024, 1), lambda b: (b, 0, 0)),
    )(y6, a6, b6)


def _conv7_kernel(g_ref, x1_ref, x2_ref, x3_ref, wg_ref, wl_ref, y_ref,
                  st_ref):
    b = pl.program_id(0)
    t = pl.program_id(1)
    base = jnp.dot(wg_ref[...], _bf16(g_ref[0]),
                   preferred_element_type=F32)
    xcat = _bf16(jnp.concatenate([x1_ref[0], x2_ref[0], x3_ref[0]], axis=0))
    y = base + jnp.dot(wl_ref[...], xcat, preferred_element_type=F32)
    y_ref[0] = y
    st = jnp.concatenate([jnp.sum(y, axis=1, keepdims=True),
                          jnp.sum(y * y, axis=1, keepdims=True)], axis=1)

    @pl.when((b == 0) & (t == 0))
    def _():
        st_ref[...] = st

    @pl.when((b != 0) | (t != 0))
    def _():
        st_ref[...] += st


def _conv7(g, x1, x2, x3, w7g, w7l):
    return pl.pallas_call(
        _conv7_kernel,
        out_shape=(
            jax.ShapeDtypeStruct((B, 512, N), F32),
            jax.ShapeDtypeStruct((512, 2), F32),
        ),
        grid=(B, NT),
        in_specs=[
            pl.BlockSpec((1, 1024, 1), lambda b, t: (b, 0, 0)),
            pl.BlockSpec((1, 64, T), lambda b, t: (b, 0, t)),
            pl.BlockSpec((1, 64, T), lambda b, t: (b, 0, t)),
            pl.BlockSpec((1, 64, T), lambda b, t: (b, 0, t)),
            pl.BlockSpec((512, 1024), lambda b, t: (0, 0)),
            pl.BlockSpec((512, 192), lambda b, t: (0, 0)),
        ],
        out_specs=(
            pl.BlockSpec((1, 512, T), lambda b, t: (b, 0, t)),
            pl.BlockSpec((512, 2), lambda b, t: (0, 0)),
        ),
    )(g, x1, x2, x3, w7g, w7l)


def _bn_conv_kernel(y_ref, a_ref, b_ref, w_ref, yo_ref, st_ref):
    b = pl.program_id(0)
    t = pl.program_id(1)
    z = _bf16(_lrelu(y_ref[0] * a_ref[...] + b_ref[...]))
    y = jnp.dot(w_ref[...], z, preferred_element_type=F32)
    yo_ref[0] = y
    st = jnp.concatenate([jnp.sum(y, axis=1, keepdims=True),
                          jnp.sum(y * y, axis=1, keepdims=True)], axis=1)

    @pl.when((b == 0) & (t == 0))
    def _():
        st_ref[...] = st

    @pl.when((b != 0) | (t != 0))
    def _():
        st_ref[...] += st


def _bn_conv(y, a, bb, w):
    cin = y.shape[1]
    cout = w.shape[0]
    return pl.pallas_call(
        _bn_conv_kernel,
        out_shape=(
            jax.ShapeDtypeStruct((B, cout, N), F32),
            jax.ShapeDtypeStruct((cout, 2), F32),
        ),
        grid=(B, NT),
        in_specs=[
            pl.BlockSpec((1, cin, T), lambda b, t: (b, 0, t)),
            pl.BlockSpec((cin, 1), lambda b, t: (0, 0)),
            pl.BlockSpec((cin, 1), lambda b, t: (0, 0)),
            pl.BlockSpec((cout, cin), lambda b, t: (0, 0)),
        ],
        out_specs=(
            pl.BlockSpec((1, cout, T), lambda b, t: (b, 0, t)),
            pl.BlockSpec((cout, 2), lambda b, t: (0, 0)),
        ),
    )(y, a, bb, w)


def _final_kernel(y_ref, a_ref, b_ref, w_ref, b9_ref, o_ref):
    z = _bf16(_lrelu(y_ref[0] * a_ref[...] + b_ref[...]))
    o_ref[0] = jnp.dot(w_ref[...], z, preferred_element_type=F32) + b9_ref[...]


def _final(y8, a8, b8, w9, b9):
    return pl.pallas_call(
        _final_kernel,
        out_shape=jax.ShapeDtypeStruct((B, 15, N), F32),
        grid=(B, NT),
        in_specs=[
            pl.BlockSpec((1, 256, T), lambda b, t: (b, 0, t)),
            pl.BlockSpec((256, 1), lambda b, t: (0, 0)),
            pl.BlockSpec((256, 1), lambda b, t: (0, 0)),
            pl.BlockSpec((15, 256), lambda b, t: (0, 0)),
            pl.BlockSpec((15, 1), lambda b, t: (0, 0)),
        ],
        out_specs=pl.BlockSpec((1, 15, T), lambda b, t: (b, 0, t)),
    )(y8, a8, b8, w9, b9)


def kernel(x_s, W1, W2, W3, W4, W5, W6, W7, W8, W9, b9,
           g1, bt1, g2, bt2, g3, bt3, g4, bt4, g5, bt5,
           g6, bt6, g7, bt7, g8, bt8):
    cnt2d = float(B * N * K)
    cnt1d = float(B * N)
    W1r, W2r, W3r, W4r, W5r, W6r, W7r, W8r, W9r = (
        _bf16(W) for W in (W1, W2, W3, W4, W5, W6, W7, W8, W9))

    idx = _knn(x_s)

    y1buf, st1 = _edge_conv1(x_s, idx, W1r)
    a1, b1 = _bn_coef(st1, g1, bt1, cnt2d)
    y2buf, st2 = _bn_conv2(y1buf, a1, b1, W2r)
    a2, b2 = _bn_coef(st2, g2, bt2, cnt2d)
    x1 = _edge_maxpool(y2buf, a2, b2)

    y3buf, st3 = _edge_conv1(x1, idx, W3r)
    a3, b3 = _bn_coef(st3, g3, bt3, cnt2d)
    y4buf, st4 = _bn_conv2(y3buf, a3, b3, W4r)
    a4, b4 = _bn_coef(st4, g4, bt4, cnt2d)
    x2 = _edge_maxpool(y4buf, a4, b4)

    y5buf, st5 = _edge_conv1(x2, idx, W5r)
    a5, b5 = _bn_coef(st5, g5, bt5, cnt2d)
    x3 = _edge_maxpool(y5buf, a5, b5)

    y6, st6 = _conv6(x1, x2, x3, W6r)
    a6, b6 = _bn_coef(st6, g6, bt6, cnt1d)
    g = _globalpool(y6, a6, b6)
    y7, st7 = _conv7(g, x1, x2, x3, W7r[:, :1024], W7r[:, 1024:])
    a7, b7 = _bn_coef(st7, g7, bt7, cnt1d)
    y8, st8 = _bn_conv(y7, a7, b7, W8r)
    a8, b8 = _bn_coef(st8, g8, bt8, cnt1d)
    out = _final(y8, a8, b8, W9r, b9[:, None])
    return (out, jnp.array([0.0], dtype=F32))


# speedup vs baseline: 2.7759x; 1.2446x over previous
import jax
import jax.numpy as jnp
from jax.experimental import pallas as pl
from jax.experimental.pallas import tpu as pltpu

B, N, K = 8, 1024, 40
EPS = 1e-5
T = 1024
NT = N // T
NEG = -jnp.inf

F32 = jnp.float32
HI = jax.lax.Precision.HIGHEST


def _lrelu(x):
    return jnp.where(x >= 0, x, 0.2 * x)


def _bf16(x):
    return x.astype(jnp.bfloat16).astype(F32)


def _knn_kernel(x_ref, idx_ref, d_ref):
    x = x_ref[0]
    xb = _bf16(x)
    xbT = xb.T
    G = xbT[:, 0:1] * xb[0:1, :]
    G = G + xbT[:, 1:2] * xb[1:2, :]
    G = G + xbT[:, 2:3] * xb[2:3, :]
    inner = -2.0 * G
    x2 = x * x
    xx_lane = jnp.sum(x2, axis=0, keepdims=True)
    xT = x.T
    xx_sub = jnp.sum(xT * xT, axis=1, keepdims=True)
    d_ref[...] = ((0.0 - xx_sub) - inner) - xx_lane
    iota = jax.lax.broadcasted_iota(jnp.int32, (N, N), 0)

    def body(k, _):
        D = d_ref[...]
        m = jnp.max(D, axis=0, keepdims=True)
        sel = jnp.min(jnp.where(D == m, iota, N), axis=0, keepdims=True)
        idx_ref[0, pl.ds(k, 1), :] = sel
        d_ref[...] = jnp.where(iota == sel, NEG, D)
        return 0

    jax.lax.fori_loop(0, K, body, 0, unroll=False)


def _knn(x_s):
    return pl.pallas_call(
        _knn_kernel,
        out_shape=jax.ShapeDtypeStruct((B, K, N), jnp.int32),
        grid=(B,),
        in_specs=[pl.BlockSpec((1, 3, N), lambda b: (b, 0, 0))],
        out_specs=pl.BlockSpec((1, K, N), lambda b: (b, 0, 0)),
        scratch_shapes=[pltpu.VMEM((N, N), F32)],
    )(x_s)


def _bn_coef(st, g, bt, cnt):
    mean = st[:, 0] / cnt
    var = st[:, 1] / cnt - mean * mean
    a = g / jnp.sqrt(var + EPS)
    b = bt - mean * a
    return a[:, None], b[:, None]


def _edge_conv1_kernel(x_ref, xt_ref, idx_ref, w1_ref, y1_ref, st_ref):
    b = pl.program_id(0)
    t = pl.program_id(1)
    x = x_ref[0]
    xt = xt_ref[0]
    W1 = w1_ref[...]
    cin = x.shape[0]

    def body(k, carry):
        s, q = carry
        ik = idx_ref[0, pl.ds(k, 1), :]
        xg = jnp.zeros((cin, T), F32)
        for g in range(N // 128):
            rel = ik - (g * 128)
            inb = (rel >= 0) & (rel < 128)
            idxb = jnp.broadcast_to(jnp.where(inb, rel, 0), (cin, T))
            gat = jnp.take_along_axis(x[:, g * 128:(g + 1) * 128],
                                      idxb, axis=1)
            xg = jnp.where(inb, gat, xg)
        ef = _bf16(jnp.concatenate([xg - xt, xt], axis=0))
        y1 = jnp.dot(W1, ef, preferred_element_type=F32)
        y1_ref[0, pl.ds(k, 1), :, :] = y1[None]
        return (s + jnp.sum(y1, axis=1, keepdims=True),
                q + jnp.sum(y1 * y1, axis=1, keepdims=True))

    s, q = jax.lax.fori_loop(0, K, body,
                             (jnp.zeros((64, 1), F32), jnp.zeros((64, 1), F32)),
                             unroll=False)
    st = jnp.concatenate([s, q], axis=1)

    @pl.when((b == 0) & (t == 0))
    def _():
        st_ref[...] = st

    @pl.when((b != 0) | (t != 0))
    def _():
        st_ref[...] += st


def _edge_conv1(x, idx, w1):
    cin = x.shape[1]
    return pl.pallas_call(
        _edge_conv1_kernel,
        out_shape=(
            jax.ShapeDtypeStruct((B, K, 64, N), F32),
            jax.ShapeDtypeStruct((64, 2), F32),
        ),
        grid=(B, NT),
        in_specs=[
            pl.BlockSpec((1, cin, N), lambda b, t: (b, 0, 0)),
            pl.BlockSpec((1, cin, T), lambda b, t: (b, 0, t)),
            pl.BlockSpec((1, K, T), lambda b, t: (b, 0, t)),
            pl.BlockSpec(w1.shape, lambda b, t: (0, 0)),
        ],
        out_specs=(
            pl.BlockSpec((1, K, 64, T), lambda b, t: (b, 0, 0, t)),
            pl.BlockSpec((64, 2), lambda b, t: (0, 0)),
        ),
    )(x, x, idx, w1)


def _bn_conv2_kernel(y1_ref, a1_ref, b1_ref, w2_ref, y2_ref, st_ref):
    b = pl.program_id(0)
    t = pl.program_id(1)
    a1 = a1_ref[...]
    b1 = b1_ref[...]
    W2 = w2_ref[...]

    def body(k, carry):
        s, q = carry
        y1 = y1_ref[0, pl.ds(k, 1), :, :][0]
        z1 = _bf16(_lrelu(y1 * a1 + b1))
        y2 = jnp.dot(W2, z1, preferred_element_type=F32)
        y2_ref[0, pl.ds(k, 1), :, :] = y2[None]
        return (s + jnp.sum(y2, axis=1, keepdims=True),
                q + jnp.sum(y2 * y2, axis=1, keepdims=True))

    s, q = jax.lax.fori_loop(0, K, body,
                             (jnp.zeros((64, 1), F32), jnp.zeros((64, 1), F32)),
                             unroll=False)
    st = jnp.concatenate([s, q], axis=1)

    @pl.when((b == 0) & (t == 0))
    def _():
        st_ref[...] = st

    @pl.when((b != 0) | (t != 0))
    def _():
        st_ref[...] += st


def _bn_conv2(y1buf, a1, b1, w2):
    return pl.pallas_call(
        _bn_conv2_kernel,
        out_shape=(
            jax.ShapeDtypeStruct((B, K, 64, N), F32),
            jax.ShapeDtypeStruct((64, 2), F32),
        ),
        grid=(B, NT),
        in_specs=[
            pl.BlockSpec((1, K, 64, T), lambda b, t: (b, 0, 0, t)),
            pl.BlockSpec((64, 1), lambda b, t: (0, 0)),
            pl.BlockSpec((64, 1), lambda b, t: (0, 0)),
            pl.BlockSpec(w2.shape, lambda b, t: (0, 0)),
        ],
        out_specs=(
            pl.BlockSpec((1, K, 64, T), lambda b, t: (b, 0, 0, t)),
            pl.BlockSpec((64, 2), lambda b, t: (0, 0)),
        ),
    )(y1buf, a1, b1, w2)


def _maxpool_kernel(y2_ref, a2_ref, b2_ref, x1_ref):
    a2 = a2_ref[...]
    b2 = b2_ref[...]

    def body(k, m):
        y2 = y2_ref[0, pl.ds(k, 1), :, :][0]
        return jnp.maximum(m, _lrelu(y2 * a2 + b2))

    x1_ref[0] = jax.lax.fori_loop(0, K, body,
                                  jnp.full((64, T), NEG, F32), unroll=False)


def _edge_maxpool(y2buf, a2, b2):
    return pl.pallas_call(
        _maxpool_kernel,
        out_shape=jax.ShapeDtypeStruct((B, 64, N), F32),
        grid=(B, NT),
        in_specs=[
            pl.BlockSpec((1, K, 64, T), lambda b, t: (b, 0, 0, t)),
            pl.BlockSpec((64, 1), lambda b, t: (0, 0)),
            pl.BlockSpec((64, 1), lambda b, t: (0, 0)),
        ],
        out_specs=pl.BlockSpec((1, 64, T), lambda b, t: (b, 0, t)),
    )(y2buf, a2, b2)


def _conv6_kernel(x1r, x2r, x3r, wr, yr, st_ref):
    b = pl.program_id(0)
    t = pl.program_id(1)
    xcat = _bf16(jnp.concatenate([x1r[0], x2r[0], x3r[0]], axis=0))
    y = jnp.dot(wr[...], xcat, preferred_element_type=F32)
    yr[0] = y
    st = jnp.concatenate([jnp.sum(y, axis=1, keepdims=True),
                          jnp.sum(y * y, axis=1, keepdims=True)], axis=1)

    @pl.when((b == 0) & (t == 0))
    def _():
        st_ref[...] = st

    @pl.when((b != 0) | (t != 0))
    def _():
        st_ref[...] += st


def _conv6(x1, x2, x3, w6):
    return pl.pallas_call(
        _conv6_kernel,
        out_shape=(
            jax.ShapeDtypeStruct((B, 1024, N), F32),
            jax.ShapeDtypeStruct((1024, 2), F32),
        ),
        grid=(B, NT),
        in_specs=[
            pl.BlockSpec((1, 64, T), lambda b, t: (b, 0, t)),
            pl.BlockSpec((1, 64, T), lambda b, t: (b, 0, t)),
            pl.BlockSpec((1, 64, T), lambda b, t: (b, 0, t)),
            pl.BlockSpec((1024, 192), lambda b, t: (0, 0)),
        ],
        out_specs=(
            pl.BlockSpec((1, 1024, T), lambda b, t: (b, 0, t)),
            pl.BlockSpec((1024, 2), lambda b, t: (0, 0)),
        ),
    )(x1, x2, x3, w6)


def _globalpool_kernel(y6_ref, a6_ref, b6_ref, g_ref):
    z = _lrelu(y6_ref[0] * a6_ref[...] + b6_ref[...])
    g_ref[0] = jnp.max(z, axis=1, keepdims=True)


def _globalpool(y6, a6, b6):
    return pl.pallas_call(
        _globalpool_kernel,
        out_shape=jax.ShapeDtypeStruct((B, 1024, 1), F32),
        grid=(B,),
        in_specs=[
            pl.BlockSpec((1, 1024, N), lambda b: (b, 0, 0)),
            pl.BlockSpec((1024, 1), lambda b: (0, 0)),
            pl.BlockSpec((1024, 1), lambda b: (0, 0)),
        ],
        out_specs=pl.BlockSpec((1, 1024, 1), lambda b: (b, 0, 0)),
    )(y6, a6, b6)


def _conv7_kernel(g_ref, x1_ref, x2_ref, x3_ref, wg_ref, wl_ref, y_ref,
                  st_ref):
    b = pl.program_id(0)
    t = pl.program_id(1)
    base = jnp.dot(wg_ref[...], _bf16(g_ref[0]),
                   preferred_element_type=F32)
    xcat = _bf16(jnp.concatenate([x1_ref[0], x2_ref[0], x3_ref[0]], axis=0))
    y = base + jnp.dot(wl_ref[...], xcat, preferred_element_type=F32)
    y_ref[0] = y
    st = jnp.concatenate([jnp.sum(y, axis=1, keepdims=True),
                          jnp.sum(y * y, axis=1, keepdims=True)], axis=1)

    @pl.when((b == 0) & (t == 0))
    def _():
        st_ref[...] = st

    @pl.when((b != 0) | (t != 0))
    def _():
        st_ref[...] += st


def _conv7(g, x1, x2, x3, w7g, w7l):
    return pl.pallas_call(
        _conv7_kernel,
        out_shape=(
            jax.ShapeDtypeStruct((B, 512, N), F32),
            jax.ShapeDtypeStruct((512, 2), F32),
        ),
        grid=(B, NT),
        in_specs=[
            pl.BlockSpec((1, 1024, 1), lambda b, t: (b, 0, 0)),
            pl.BlockSpec((1, 64, T), lambda b, t: (b, 0, t)),
            pl.BlockSpec((1, 64, T), lambda b, t: (b, 0, t)),
            pl.BlockSpec((1, 64, T), lambda b, t: (b, 0, t)),
            pl.BlockSpec((512, 1024), lambda b, t: (0, 0)),
            pl.BlockSpec((512, 192), lambda b, t: (0, 0)),
        ],
        out_specs=(
            pl.BlockSpec((1, 512, T), lambda b, t: (b, 0, t)),
            pl.BlockSpec((512, 2), lambda b, t: (0, 0)),
        ),
    )(g, x1, x2, x3, w7g, w7l)


def _bn_conv_kernel(y_ref, a_ref, b_ref, w_ref, yo_ref, st_ref):
    b = pl.program_id(0)
    t = pl.program_id(1)
    z = _bf16(_lrelu(y_ref[0] * a_ref[...] + b_ref[...]))
    y = jnp.dot(w_ref[...], z, preferred_element_type=F32)
    yo_ref[0] = y
    st = jnp.concatenate([jnp.sum(y, axis=1, keepdims=True),
                          jnp.sum(y * y, axis=1, keepdims=True)], axis=1)

    @pl.when((b == 0) & (t == 0))
    def _():
        st_ref[...] = st

    @pl.when((b != 0) | (t != 0))
    def _():
        st_ref[...] += st


def _bn_conv(y, a, bb, w):
    cin = y.shape[1]
    cout = w.shape[0]
    return pl.pallas_call(
        _bn_conv_kernel,
        out_shape=(
            jax.ShapeDtypeStruct((B, cout, N), F32),
            jax.ShapeDtypeStruct((cout, 2), F32),
        ),
        grid=(B, NT),
        in_specs=[
            pl.BlockSpec((1, cin, T), lambda b, t: (b, 0, t)),
            pl.BlockSpec((cin, 1), lambda b, t: (0, 0)),
            pl.BlockSpec((cin, 1), lambda b, t: (0, 0)),
            pl.BlockSpec((cout, cin), lambda b, t: (0, 0)),
        ],
        out_specs=(
            pl.BlockSpec((1, cout, T), lambda b, t: (b, 0, t)),
            pl.BlockSpec((cout, 2), lambda b, t: (0, 0)),
        ),
    )(y, a, bb, w)


def _final_kernel(y_ref, a_ref, b_ref, w_ref, b9_ref, o_ref):
    z = _bf16(_lrelu(y_ref[0] * a_ref[...] + b_ref[...]))
    o_ref[0] = jnp.dot(w_ref[...], z, preferred_element_type=F32) + b9_ref[...]


def _final(y8, a8, b8, w9, b9):
    return pl.pallas_call(
        _final_kernel,
        out_shape=jax.ShapeDtypeStruct((B, 15, N), F32),
        grid=(B, NT),
        in_specs=[
            pl.BlockSpec((1, 256, T), lambda b, t: (b, 0, t)),
            pl.BlockSpec((256, 1), lambda b, t: (0, 0)),
            pl.BlockSpec((256, 1), lambda b, t: (0, 0)),
            pl.BlockSpec((15, 256), lambda b, t: (0, 0)),
            pl.BlockSpec((15, 1), lambda b, t: (0, 0)),
        ],
        out_specs=pl.BlockSpec((1, 15, T), lambda b, t: (b, 0, t)),
    )(y8, a8, b8, w9, b9)


def kernel(x_s, W1, W2, W3, W4, W5, W6, W7, W8, W9, b9,
           g1, bt1, g2, bt2, g3, bt3, g4, bt4, g5, bt5,
           g6, bt6, g7, bt7, g8, bt8):
    cnt2d = float(B * N * K)
    cnt1d = float(B * N)
    W1r, W2r, W3r, W4r, W5r, W6r, W7r, W8r, W9r = (
        _bf16(W) for W in (W1, W2, W3, W4, W5, W6, W7, W8, W9))

    idx = _knn(x_s)

    y1buf, st1 = _edge_conv1(x_s, idx, W1r)
    a1, b1 = _bn_coef(st1, g1, bt1, cnt2d)
    y2buf, st2 = _bn_conv2(y1buf, a1, b1, W2r)
    a2, b2 = _bn_coef(st2, g2, bt2, cnt2d)
    x1 = _edge_maxpool(y2buf, a2, b2)

    y3buf, st3 = _edge_conv1(x1, idx, W3r)
    a3, b3 = _bn_coef(st3, g3, bt3, cnt2d)
    y4buf, st4 = _bn_conv2(y3buf, a3, b3, W4r)
    a4, b4 = _bn_coef(st4, g4, bt4, cnt2d)
    x2 = _edge_maxpool(y4buf, a4, b4)

    y5buf, st5 = _edge_conv1(x2, idx, W5r)
    a5, b5 = _bn_coef(st5, g5, bt5, cnt2d)
    x3 = _edge_maxpool(y5buf, a5, b5)

    y6, st6 = _conv6(x1, x2, x3, W6r)
    a6, b6 = _bn_coef(st6, g6, bt6, cnt1d)
    g = _globalpool(y6, a6, b6)
    y7, st7 = _conv7(g, x1, x2, x3, W7r[:, :1024], W7r[:, 1024:])
    a7, b7 = _bn_coef(st7, g7, bt7, cnt1d)
    y8, st8 = _bn_conv(y7, a7, b7, W8r)
    a8, b8 = _bn_coef(st8, g8, bt8, cnt1d)
    out = _final(y8, a8, b8, W9r, b9[:, None])
    return (out, jnp.array([0.0], dtype=F32))
